# baseline (speedup 1.0000x reference)
import functools

import jax
import jax.numpy as jnp
from jax import lax
from jax.experimental import pallas as pl
from jax.experimental.pallas import tpu as pltpu
from jax.experimental.pallas import tpu_sc as plsc

VOCAB = 1000000
D_MODEL = 64
B_ROWS = 4096
T_COLS = 200

_info = plsc.get_sparse_core_info()
NC = _info.num_cores
NS = _info.num_subcores
NW = NC * NS
BL = B_ROWS // NW
NB = 3
NG = T_COLS // NB
NFULL = VOCAB // 128
KB_BASE = NFULL // NW
KB_EXTRA = NFULL % NW


def _make_repack():
    mesh = plsc.VectorSubcoreMesh(core_axis_name="c", subcore_axis_name="s")

    scratch = {
        "bufs": [pltpu.VMEM((D_MODEL, 128), jnp.float32) for _ in range(2)],
        "slabs": [pltpu.VMEM((D_MODEL, 133), jnp.float32) for _ in range(2)],
        "gsem": pltpu.SemaphoreType.DMA((2,)),
        "wsem": pltpu.SemaphoreType.DMA((2,)),
    }

    @functools.partial(
        pl.kernel,
        mesh=mesh,
        out_type=jax.ShapeDtypeStruct((VOCAB // 2, 128), jnp.float32),
        scratch_types=scratch,
        compiler_params=pltpu.CompilerParams(needs_layout_passes=False),
    )
    def repack_kernel(tab_hbm, tail_hbm, out_hbm, bufs, slabs, gsem, wsem):
        wid = lax.axis_index("s") * NC + lax.axis_index("c")
        nblk = KB_BASE + jnp.where(wid < KB_EXTRA, 1, 0)
        base = wid * KB_BASE + jnp.minimum(wid, KB_EXTRA)

        iota = lax.iota(jnp.int32, 16)

        def fire(slot, j):
            pltpu.async_copy(tab_hbm.at[:, pl.ds(j * 128, 128)], bufs[slot],
                             gsem.at[slot])

        def gwait(slot):
            pltpu.make_async_copy(tab_hbm.at[:, pl.ds(0, 128)], bufs[slot],
                                  gsem.at[slot]).wait()

        def transpose(slot):
            buf, slab = bufs[slot], slabs[slot]

            @plsc.parallel_loop(0, D_MODEL, step=1, unroll=8)
            def _(d):
                for g in range(8):
                    rows = (g * 16 + iota) // 2
                    cols = ((g * 16 + iota) % 2) * 64
                    v = buf[d, pl.ds(g * 16, 16)]
                    plsc.store_scatter(slab, [rows, cols + d], v)

        def wb(slot, j):
            pltpu.async_copy(slabs[slot].at[:, pl.ds(0, 128)],
                             out_hbm.at[pl.ds(j * 64, 64)], wsem.at[slot])

        def wwait(slot):
            pltpu.make_async_copy(slabs[slot].at[:, pl.ds(0, 128)],
                                  out_hbm.at[pl.ds(0, 64)],
                                  wsem.at[slot]).wait()

        fire(0, base)

        def body(i, carry):
            for s in range(2):
                jj = base + i * 2 + s

                @pl.when(jj < base + nblk)
                def _():
                    nxt = jj + 1
                    other = 1 - s

                    @pl.when(nxt < base + nblk)
                    def _():
                        fire(other, nxt)

                    gwait(s)

                    @pl.when(i * 2 + s >= 2)
                    def _():
                        wwait(s)

                    transpose(s)
                    wb(s, jj)
            return carry

        lax.fori_loop(0, (KB_BASE + 2) // 2, body, 0, unroll=False)
        for s in range(2):
            @pl.when(KB_BASE + jnp.where(wid < KB_EXTRA, 1, 0) > s)
            def _():
                wwait(s)

        @pl.when(wid == NW - 1)
        def _():
            pltpu.sync_copy(tail_hbm, bufs[0])
            transpose(0)
            pltpu.sync_copy(slabs[0].at[:, pl.ds(0, 128)],
                            out_hbm.at[pl.ds((VOCAB - 128) // 2, 64)])

    return repack_kernel


def _make_gather():
    mesh = plsc.VectorSubcoreMesh(core_axis_name="c", subcore_axis_name="s")

    scratch = {
        "idx_v": pltpu.VMEM((T_COLS, BL), jnp.int32),
        "bufs": [pltpu.VMEM((BL, D_MODEL), jnp.float32) for _ in range(NB)],
        "slabs": [pltpu.VMEM((8, 8, 133), jnp.float32) for _ in range(NB)],
        "gsem": pltpu.SemaphoreType.DMA((NB,)),
        "wsem": pltpu.SemaphoreType.DMA((NB,)),
    }

    @functools.partial(
        pl.kernel,
        mesh=mesh,
        out_type=jax.ShapeDtypeStruct((T_COLS, 8, NW, 8, BL), jnp.float32),
        scratch_types=scratch,
        compiler_params=pltpu.CompilerParams(
            use_tc_tiling_on_sc=False, needs_layout_passes=False),
    )
    def gather_kernel(idx_hbm, table_hbm, out_hbm, idx_v, bufs, slabs,
                      gsem, wsem):
        wid = lax.axis_index("s") * NC + lax.axis_index("c")
        pltpu.sync_copy(idx_hbm.at[:, pl.ds(wid * BL, BL)], idx_v)

        iota = lax.iota(jnp.int32, 16)

        def fire(slot, t):
            pltpu.async_copy(table_hbm.at[idx_v.at[t]], bufs[slot],
                             gsem.at[slot])

        def gather_wait(slot):
            pltpu.make_async_copy(table_hbm.at[idx_v.at[0]], bufs[slot],
                                  gsem.at[slot]).wait()

        def transpose(slot):
            buf, slab = bufs[slot], slabs[slot]

            @plsc.parallel_loop(0, BL, step=1, unroll=8)
            def _(bl):
                blv = jnp.full((16,), bl, jnp.int32)
                for k in range(4):
                    hi = (k * 16 + iota) // 8
                    lo = (k * 16 + iota) % 8
                    v = buf[bl, pl.ds(k * 16, 16)]
                    plsc.store_scatter(slab, [hi, lo, blv], v)

        def wb_start(slot, t):
            pltpu.async_copy(slabs[slot].at[:, :, pl.ds(0, BL)],
                             out_hbm.at[t].at[:, wid], wsem.at[slot])

        def wb_wait(slot):
            pltpu.make_async_copy(slabs[slot].at[:, :, pl.ds(0, BL)],
                                  out_hbm.at[0].at[:, 0],
                                  wsem.at[slot]).wait()

        for b in range(NB - 1):
            fire(b, b)

        def group(g, carry):
            for b in range(NB):
                t = g * NB + b
                t_pre = t + NB - 1
                slot_pre = (b + NB - 1) % NB

                @pl.when(t_pre < T_COLS)
                def _():
                    fire(slot_pre, t_pre)

                gather_wait(b)

                @pl.when(t >= NB)
                def _():
                    wb_wait(b)

                transpose(b)
                wb_start(b, t)
            return carry

        lax.fori_loop(0, NG, group, 0, unroll=False)

        for r in range(T_COLS % NB):
            t = NG * NB + r
            gather_wait(r)
            wb_wait(r)
            transpose(r)
            wb_start(r, t)
        for b in range(NB):
            wb_wait(b)

    return gather_kernel


_repack = _make_repack()
_gather = _make_gather()


def kernel(token_ids, table):
    idx_t = token_ids.T
    packed = _repack(table.T, table[VOCAB - 128:, :].T)
    table_rows = packed.reshape(VOCAB, D_MODEL)
    out5 = _gather(idx_t, table_rows)
    out = jnp.transpose(out5, (2, 4, 0, 1, 3)).reshape(B_ROWS, T_COLS, D_MODEL)
    return out


# speedup vs baseline: 2.3777x; 2.3777x over previous
import functools

import jax
import jax.numpy as jnp
from jax import lax
from jax.experimental import pallas as pl
from jax.experimental.pallas import tpu as pltpu
from jax.experimental.pallas import tpu_sc as plsc

VOCAB = 1000000
D_MODEL = 64
B_ROWS = 4096
T_COLS = 200

_info = plsc.get_sparse_core_info()
NC = _info.num_cores
NS = _info.num_subcores
NW = NC * NS
BL = B_ROWS // NW
NB = 3
NG = T_COLS // NB
NFULL = VOCAB // 128
KB_BASE = NFULL // NW
KB_EXTRA = NFULL % NW


def _make_repack():
    mesh = plsc.VectorSubcoreMesh(core_axis_name="c", subcore_axis_name="s")

    scratch = {
        "bufs": [pltpu.VMEM((D_MODEL, 128), jnp.float32) for _ in range(2)],
        "slabs": [pltpu.VMEM((D_MODEL, 133), jnp.float32) for _ in range(2)],
        "gsem": pltpu.SemaphoreType.DMA((2,)),
        "wsem": pltpu.SemaphoreType.DMA((2,)),
    }

    @functools.partial(
        pl.kernel,
        mesh=mesh,
        out_type=jax.ShapeDtypeStruct((VOCAB // 2, 128), jnp.float32),
        scratch_types=scratch,
        compiler_params=pltpu.CompilerParams(needs_layout_passes=False),
    )
    def repack_kernel(tab_hbm, tail_hbm, out_hbm, bufs, slabs, gsem, wsem):
        wid = lax.axis_index("s") * NC + lax.axis_index("c")
        nblk = KB_BASE + jnp.where(wid < KB_EXTRA, 1, 0)
        base = wid * KB_BASE + jnp.minimum(wid, KB_EXTRA)

        iota = lax.iota(jnp.int32, 16)

        def fire(slot, j):
            pltpu.async_copy(tab_hbm.at[:, pl.ds(j * 128, 128)], bufs[slot],
                             gsem.at[slot])

        def gwait(slot):
            pltpu.make_async_copy(tab_hbm.at[:, pl.ds(0, 128)], bufs[slot],
                                  gsem.at[slot]).wait()

        zero = jnp.zeros((16,), jnp.int32)
        bases = [((g * 16 + iota) // 2) * 133 + ((g * 16 + iota) % 2) * 64
                 for g in range(8)]

        def transpose(slot):
            buf, slab = bufs[slot], slabs[slot]

            @plsc.parallel_loop(0, D_MODEL, step=1, unroll=8)
            def _(d):
                for g in range(8):
                    v = buf[d, pl.ds(g * 16, 16)]
                    plsc.store_scatter(slab, [zero, bases[g] + d], v)

        def wb(slot, j):
            pltpu.async_copy(slabs[slot].at[:, pl.ds(0, 128)],
                             out_hbm.at[pl.ds(j * 64, 64)], wsem.at[slot])

        def wwait(slot):
            pltpu.make_async_copy(slabs[slot].at[:, pl.ds(0, 128)],
                                  out_hbm.at[pl.ds(0, 64)],
                                  wsem.at[slot]).wait()

        fire(0, base)

        def body(i, carry):
            for s in range(2):
                jj = base + i * 2 + s

                @pl.when(jj < base + nblk)
                def _():
                    nxt = jj + 1
                    other = 1 - s

                    @pl.when(nxt < base + nblk)
                    def _():
                        fire(other, nxt)

                    gwait(s)

                    @pl.when(i * 2 + s >= 2)
                    def _():
                        wwait(s)

                    transpose(s)
                    wb(s, jj)
            return carry

        lax.fori_loop(0, (KB_BASE + 2) // 2, body, 0, unroll=False)
        for s in range(2):
            @pl.when(KB_BASE + jnp.where(wid < KB_EXTRA, 1, 0) > s)
            def _():
                wwait(s)

        @pl.when(wid == NW - 1)
        def _():
            pltpu.sync_copy(tail_hbm, bufs[0])
            transpose(0)
            pltpu.sync_copy(slabs[0].at[:, pl.ds(0, 128)],
                            out_hbm.at[pl.ds((VOCAB - 128) // 2, 64)])

    return repack_kernel


def _make_gather():
    mesh = plsc.VectorSubcoreMesh(core_axis_name="c", subcore_axis_name="s")

    scratch = {
        "idx_v": pltpu.VMEM((T_COLS, BL), jnp.int32),
        "bufs": [pltpu.VMEM((BL, D_MODEL), jnp.float32) for _ in range(NB)],
        "slabs": [pltpu.VMEM((8, 8, 133), jnp.float32) for _ in range(NB)],
        "gsem": pltpu.SemaphoreType.DMA((NB,)),
        "wsem": pltpu.SemaphoreType.DMA((NB,)),
    }

    @functools.partial(
        pl.kernel,
        mesh=mesh,
        out_type=jax.ShapeDtypeStruct((T_COLS, 8, NW, 8, BL), jnp.float32),
        scratch_types=scratch,
        compiler_params=pltpu.CompilerParams(
            use_tc_tiling_on_sc=False, needs_layout_passes=False),
    )
    def gather_kernel(idx_hbm, table_hbm, out_hbm, idx_v, bufs, slabs,
                      gsem, wsem):
        wid = lax.axis_index("s") * NC + lax.axis_index("c")
        pltpu.sync_copy(idx_hbm.at[:, pl.ds(wid * BL, BL)], idx_v)

        iota = lax.iota(jnp.int32, 16)

        def fire(slot, t):
            pltpu.async_copy(table_hbm.at[idx_v.at[t]], bufs[slot],
                             gsem.at[slot])

        def gather_wait(slot):
            pltpu.make_async_copy(table_hbm.at[idx_v.at[0]], bufs[slot],
                                  gsem.at[slot]).wait()

        def transpose(slot):
            buf, slab = bufs[slot], slabs[slot]

            @plsc.parallel_loop(0, BL, step=1, unroll=8)
            def _(bl):
                blv = jnp.full((16,), bl, jnp.int32)
                for k in range(4):
                    hi = (k * 16 + iota) // 8
                    lo = (k * 16 + iota) % 8
                    v = buf[bl, pl.ds(k * 16, 16)]
                    plsc.store_scatter(slab, [hi, lo, blv], v)

        def wb_start(slot, t):
            pltpu.async_copy(slabs[slot].at[:, :, pl.ds(0, BL)],
                             out_hbm.at[t].at[:, wid], wsem.at[slot])

        def wb_wait(slot):
            pltpu.make_async_copy(slabs[slot].at[:, :, pl.ds(0, BL)],
                                  out_hbm.at[0].at[:, 0],
                                  wsem.at[slot]).wait()

        for b in range(NB - 1):
            fire(b, b)

        def group(g, carry):
            for b in range(NB):
                t = g * NB + b
                t_pre = t + NB - 1
                slot_pre = (b + NB - 1) % NB

                @pl.when(t_pre < T_COLS)
                def _():
                    fire(slot_pre, t_pre)

                gather_wait(b)

                @pl.when(t >= NB)
                def _():
                    wb_wait(b)

                transpose(b)
                wb_start(b, t)
            return carry

        lax.fori_loop(0, NG, group, 0, unroll=False)

        for r in range(T_COLS % NB):
            t = NG * NB + r
            gather_wait(r)
            wb_wait(r)
            transpose(r)
            wb_start(r, t)
        for b in range(NB):
            wb_wait(b)

    return gather_kernel


_repack = _make_repack()
_gather = _make_gather()


def kernel(token_ids, table):
    idx_t = token_ids.T
    packed = _repack(table.T, table[VOCAB - 128:, :].T)
    table_rows = packed.reshape(VOCAB, D_MODEL)
    out5 = _gather(idx_t, table_rows)
    out = jnp.transpose(out5, (2, 4, 0, 1, 3)).reshape(B_ROWS, T_COLS, D_MODEL)
    return out


# speedup vs baseline: 2.3857x; 1.0034x over previous
import functools

import jax
import jax.numpy as jnp
from jax import lax
from jax.experimental import pallas as pl
from jax.experimental.pallas import tpu as pltpu
from jax.experimental.pallas import tpu_sc as plsc

VOCAB = 1000000
D_MODEL = 64
B_ROWS = 4096
T_COLS = 200

_info = plsc.get_sparse_core_info()
NC = _info.num_cores
NS = _info.num_subcores
NW = NC * NS
BL = B_ROWS // NW
NB = 3
NG = T_COLS // NB
NFULL = VOCAB // 128
KB_BASE = NFULL // NW
KB_EXTRA = NFULL % NW


def _make_repack():
    mesh = plsc.VectorSubcoreMesh(core_axis_name="c", subcore_axis_name="s")

    scratch = {
        "bufs": [pltpu.VMEM((D_MODEL, 128), jnp.float32) for _ in range(2)],
        "slabs": [pltpu.VMEM((D_MODEL, 133), jnp.float32) for _ in range(2)],
        "gsem": pltpu.SemaphoreType.DMA((2,)),
        "wsem": pltpu.SemaphoreType.DMA((2,)),
    }

    @functools.partial(
        pl.kernel,
        mesh=mesh,
        out_type=jax.ShapeDtypeStruct((VOCAB // 2, 128), jnp.float32),
        scratch_types=scratch,
        compiler_params=pltpu.CompilerParams(needs_layout_passes=False),
    )
    def repack_kernel(tab_hbm, tail_hbm, out_hbm, bufs, slabs, gsem, wsem):
        wid = lax.axis_index("s") * NC + lax.axis_index("c")
        nblk = KB_BASE + jnp.where(wid < KB_EXTRA, 1, 0)
        base = wid * KB_BASE + jnp.minimum(wid, KB_EXTRA)

        iota = lax.iota(jnp.int32, 16)

        def fire(slot, j):
            pltpu.async_copy(tab_hbm.at[:, pl.ds(j * 128, 128)], bufs[slot],
                             gsem.at[slot])

        def gwait(slot):
            pltpu.make_async_copy(tab_hbm.at[:, pl.ds(0, 128)], bufs[slot],
                                  gsem.at[slot]).wait()

        zero = jnp.zeros((16,), jnp.int32)
        bases = [((g * 16 + iota) // 2) * 133 + ((g * 16 + iota) % 2) * 64
                 for g in range(8)]

        def transpose(slot):
            buf, slab = bufs[slot], slabs[slot]

            @plsc.parallel_loop(0, D_MODEL, step=1, unroll=8)
            def _(d):
                for g in range(8):
                    v = buf[d, pl.ds(g * 16, 16)]
                    plsc.store_scatter(slab, [zero, bases[g] + d], v)

        def wb(slot, j):
            pltpu.async_copy(slabs[slot].at[:, pl.ds(0, 128)],
                             out_hbm.at[pl.ds(j * 64, 64)], wsem.at[slot])

        def wwait(slot):
            pltpu.make_async_copy(slabs[slot].at[:, pl.ds(0, 128)],
                                  out_hbm.at[pl.ds(0, 64)],
                                  wsem.at[slot]).wait()

        fire(0, base)

        def body(i, carry):
            for s in range(2):
                jj = base + i * 2 + s

                @pl.when(jj < base + nblk)
                def _():
                    nxt = jj + 1
                    other = 1 - s

                    @pl.when(nxt < base + nblk)
                    def _():
                        fire(other, nxt)

                    gwait(s)

                    @pl.when(i * 2 + s >= 2)
                    def _():
                        wwait(s)

                    transpose(s)
                    wb(s, jj)
            return carry

        lax.fori_loop(0, (KB_BASE + 2) // 2, body, 0, unroll=False)
        for s in range(2):
            @pl.when(KB_BASE + jnp.where(wid < KB_EXTRA, 1, 0) > s)
            def _():
                wwait(s)

        @pl.when(wid == NW - 1)
        def _():
            pltpu.sync_copy(tail_hbm, bufs[0])
            transpose(0)
            pltpu.sync_copy(slabs[0].at[:, pl.ds(0, 128)],
                            out_hbm.at[pl.ds((VOCAB - 128) // 2, 64)])

    return repack_kernel


def _make_gather():
    mesh = plsc.VectorSubcoreMesh(core_axis_name="c", subcore_axis_name="s")

    scratch = {
        "idx_v": pltpu.VMEM((T_COLS, BL), jnp.int32),
        "bufs": [pltpu.VMEM((BL, D_MODEL), jnp.float32) for _ in range(NB)],
        "slabs": [pltpu.VMEM((8, 8, 133), jnp.float32) for _ in range(NB)],
        "gsem": pltpu.SemaphoreType.DMA((NB,)),
        "wsem": pltpu.SemaphoreType.DMA((NB,)),
    }

    @functools.partial(
        pl.kernel,
        mesh=mesh,
        out_type=jax.ShapeDtypeStruct((T_COLS, 8, NW, 8, BL), jnp.float32),
        scratch_types=scratch,
        compiler_params=pltpu.CompilerParams(
            use_tc_tiling_on_sc=False, needs_layout_passes=False),
    )
    def gather_kernel(idx_hbm, table_hbm, out_hbm, idx_v, bufs, slabs,
                      gsem, wsem):
        wid = lax.axis_index("s") * NC + lax.axis_index("c")
        pltpu.sync_copy(idx_hbm.at[:, pl.ds(wid * BL, BL)], idx_v)

        iota = lax.iota(jnp.int32, 16)

        def fire(slot, t):
            pltpu.async_copy(table_hbm.at[idx_v.at[t]], bufs[slot],
                             gsem.at[slot])

        def gather_wait(slot):
            pltpu.make_async_copy(table_hbm.at[idx_v.at[0]], bufs[slot],
                                  gsem.at[slot]).wait()

        zero = jnp.zeros((16,), jnp.int32)
        bases = [(k * 16 + iota) * 133 for k in range(4)]

        def transpose(slot):
            buf, slab = bufs[slot], slabs[slot]

            @plsc.parallel_loop(0, BL, step=1, unroll=8)
            def _(bl):
                for k in range(4):
                    v = buf[bl, pl.ds(k * 16, 16)]
                    plsc.store_scatter(slab, [zero, zero, bases[k] + bl], v)

        def wb_start(slot, t):
            pltpu.async_copy(slabs[slot].at[:, :, pl.ds(0, BL)],
                             out_hbm.at[t].at[:, wid], wsem.at[slot])

        def wb_wait(slot):
            pltpu.make_async_copy(slabs[slot].at[:, :, pl.ds(0, BL)],
                                  out_hbm.at[0].at[:, 0],
                                  wsem.at[slot]).wait()

        for b in range(NB - 1):
            fire(b, b)

        def group(g, carry):
            for b in range(NB):
                t = g * NB + b
                t_pre = t + NB - 1
                slot_pre = (b + NB - 1) % NB

                @pl.when(t_pre < T_COLS)
                def _():
                    fire(slot_pre, t_pre)

                gather_wait(b)

                @pl.when(t >= NB)
                def _():
                    wb_wait(b)

                transpose(b)
                wb_start(b, t)
            return carry

        lax.fori_loop(0, NG, group, 0, unroll=False)

        for r in range(T_COLS % NB):
            t = NG * NB + r
            gather_wait(r)
            wb_wait(r)
            transpose(r)
            wb_start(r, t)
        for b in range(NB):
            wb_wait(b)

    return gather_kernel


_repack = _make_repack()
_gather = _make_gather()


def kernel(token_ids, table):
    idx_t = token_ids.T
    packed = _repack(table.T, table[VOCAB - 128:, :].T)
    table_rows = packed.reshape(VOCAB, D_MODEL)
    out5 = _gather(idx_t, table_rows)
    out = jnp.transpose(out5, (2, 4, 0, 1, 3)).reshape(B_ROWS, T_COLS, D_MODEL)
    return out
